# Initial kernel scaffold; baseline (speedup 1.0000x reference)
import jax
import jax.numpy as jnp
from jax.experimental import pallas as pl


def kernel(mask_tuple, mask_idx, mask_attrs, tables):
    raise NotImplementedError("write your pallas kernel here")


# speedup vs baseline: 1.2943x; 1.2943x over previous
import functools

import jax
import jax.numpy as jnp
from jax.experimental import pallas as pl
from jax.experimental.pallas import tpu as pltpu
from jax.experimental.pallas import tpu_sc as plsc

_WINDOW = 128


def _gather_two(flat_table, idx_tuple, idx_attr):
    n1 = idx_tuple.shape[0]
    n2 = idx_attr.shape[0]
    d = flat_table.shape[1]
    mesh = plsc.VectorSubcoreMesh(core_axis_name="c", subcore_axis_name="s")

    @functools.partial(
        pl.kernel,
        out_type=(
            jax.ShapeDtypeStruct((n1, d), flat_table.dtype),
            jax.ShapeDtypeStruct((n2, d), flat_table.dtype),
        ),
        mesh=mesh,
        compiler_params=pltpu.CompilerParams(use_tc_tiling_on_sc=False),
    )
    def k(tab_hbm, i1_hbm, i2_hbm, o1_hbm, o2_hbm):
        def body(i_vmem, o_vmem):
            pltpu.sync_copy(tab_hbm.at[i_vmem.at[0]], o_vmem)

        pltpu.emit_pipeline(
            body,
            grid=(n1 // _WINDOW,),
            in_specs=[pl.BlockSpec((1, _WINDOW), lambda i: (0, i))],
            out_specs=[pl.BlockSpec((_WINDOW, d), lambda i: (i, 0))],
            core_axis_name=("c", "s"),
            dimension_semantics=(pltpu.PARALLEL,),
        )(i1_hbm, o1_hbm)

        pltpu.emit_pipeline(
            body,
            grid=(n2 // _WINDOW,),
            in_specs=[pl.BlockSpec((1, _WINDOW), lambda i: (0, i))],
            out_specs=[pl.BlockSpec((_WINDOW, d), lambda i: (i, 0))],
            core_axis_name=("c", "s"),
            dimension_semantics=(pltpu.PARALLEL,),
        )(i2_hbm, o2_hbm)

    return k(flat_table, idx_tuple.reshape(1, n1), idx_attr.reshape(1, n2))


def kernel(mask_tuple, mask_idx, mask_attrs, tables):
    num_attrs, vocab, d = tables.shape
    batch = mask_tuple.shape[0]
    flat_table = tables.reshape(num_attrs * vocab, d)
    offs = (jnp.arange(num_attrs, dtype=jnp.int32) * vocab)[None, :]
    idx_tuple = (mask_tuple + offs).reshape(-1)
    base = jnp.asarray(mask_idx, jnp.int32) * vocab
    idx_attr = (mask_attrs + base).reshape(-1)
    o1, o2 = _gather_two(flat_table, idx_tuple, idx_attr)
    return (o1.reshape(batch, num_attrs * d), o2)
